# baseline (speedup 1.0000x reference)
import functools

import jax
import jax.numpy as jnp
from jax import lax
from jax.experimental import pallas as pl
from jax.experimental.pallas import tpu as pltpu
from jax.experimental.pallas import tpu_sc as plsc

V = 50000
EXT = 51000
B = 128
T = 2048
K = 8
VP = 50048
NCH = VP // 128
NCH4 = 392
NG = NCH4 // 4
RB = 8
NEG = -1e30
BIGID = 2**30

NC, NS = 2, 16
NW = NC * NS
RPW = B // NW
ACC_W = 51200


def _sc_scatter_body(attn_hbm, ids_hbm, t_hbm,
                     ids0, ids1, attn0, attn1, loc_v, t0, t1, z_v,
                     sem_in0, sem_in1, sem_o0, sem_o1, acc):
    ids_v = (ids0, ids1)
    attn_v = (attn0, attn1)
    t_v = (t0, t1)
    sem_in = (sem_in0, sem_in1)
    sem_o = (sem_o0, sem_o1)
    cid = lax.axis_index("c")
    sid = lax.axis_index("s")
    wid = sid * NC + cid
    base = sid * ACC_W

    def zf1(k, c):
        z_v[pl.ds(k * 16, 16)] = jnp.zeros((16,), jnp.float32)
        return c
    lax.fori_loop(0, T // 16, zf1, 0)

    def roff(i):
        return pl.multiple_of((wid * RPW + i) * T, T)

    pend_in = [
        (pltpu.async_copy(ids_hbm.at[pl.ds(roff(0), T)], ids_v[0], sem_in[0]),
         pltpu.async_copy(attn_hbm.at[pl.ds(roff(0), T)], attn_v[0], sem_in[0])),
        None,
    ]
    pend_out = [None, None]

    for i in range(RPW):
        b = i % 2
        for cp in pend_in[b]:
            cp.wait()
        if i + 1 < RPW:
            pend_in[1 - b] = (
                pltpu.async_copy(ids_hbm.at[pl.ds(roff(i + 1), T)],
                                 ids_v[1 - b], sem_in[1 - b]),
                pltpu.async_copy(attn_hbm.at[pl.ds(roff(i + 1), T)],
                                 attn_v[1 - b], sem_in[1 - b]),
            )
        if pend_out[b] is not None:
            pend_out[b].wait()

        def ck(k, cc):
            sl = pl.ds(k * 16, 16)
            loc_v[sl] = ids_v[b][sl] + base
            return cc
        lax.fori_loop(0, T // 16, ck, 0)

        pltpu.sync_copy(z_v, acc.at[loc_v])
        pltpu.sync_copy(attn_v[b], acc.at[loc_v], add=True)
        pltpu.sync_copy(acc.at[loc_v], t_v[b])
        pend_out[b] = pltpu.async_copy(t_v[b], t_hbm.at[pl.ds(roff(i), T)],
                                       sem_o[b])

    for po in pend_out:
        if po is not None:
            po.wait()


def _sc_gather_body(vocab_hbm, ids_hbm, g_hbm,
                    ids0, ids1, gid_v, vtile, g0, g1,
                    sem_in0, sem_in1, sem_vt, sem_g, sem_o0, sem_o1, vrowS):
    ids_v = (ids0, ids1)
    g_v = (g0, g1)
    sem_in = (sem_in0, sem_in1)
    sem_o = (sem_o0, sem_o1)
    cid = lax.axis_index("c")
    sid = lax.axis_index("s")
    wid = sid * NC + cid
    vbase = pl.multiple_of(sid * V, 8)

    def roff(i):
        return pl.multiple_of((wid * RPW + i) * T, T)

    pend_in = [
        pltpu.async_copy(ids_hbm.at[pl.ds(roff(0), T)], ids_v[0], sem_in[0]),
        None,
    ]
    pend_out = [None, None]
    pend_vt = pltpu.async_copy(vocab_hbm.at[wid * RPW], vtile, sem_vt)

    for i in range(RPW):
        b = i % 2
        pend_in[b].wait()
        if i + 1 < RPW:
            pend_in[1 - b] = pltpu.async_copy(
                ids_hbm.at[pl.ds(roff(i + 1), T)], ids_v[1 - b], sem_in[1 - b])

        def ck(k, cc):
            sl = pl.ds(k * 16, 16)
            gid_v[sl] = jnp.minimum(ids_v[b][sl], V - 1) + vbase
            return cc
        lax.fori_loop(0, T // 16, ck, 0)

        pend_vt.wait()
        pltpu.sync_copy(vtile, vrowS.at[pl.ds(vbase, V)])
        if i + 1 < RPW:
            pend_vt = pltpu.async_copy(vocab_hbm.at[wid * RPW + i + 1],
                                       vtile, sem_vt)
        if pend_out[b] is not None:
            pend_out[b].wait()
        pltpu.sync_copy(vrowS.at[gid_v], g_v[b])
        pend_out[b] = pltpu.async_copy(g_v[b], g_hbm.at[pl.ds(roff(i), T)],
                                       sem_o[b])

    for po in pend_out:
        if po is not None:
            po.wait()


@functools.lru_cache(maxsize=1)
def _make_sc_calls():
  mesh = plsc.VectorSubcoreMesh(core_axis_name="c", subcore_axis_name="s")
  scatter = functools.partial(
    pl.kernel,
    mesh=mesh,
    out_type=[jax.ShapeDtypeStruct((B * T,), jnp.float32)],
    scratch_types=[
        pltpu.VMEM((T,), jnp.int32),
        pltpu.VMEM((T,), jnp.int32),
        pltpu.VMEM((T,), jnp.float32),
        pltpu.VMEM((T,), jnp.float32),
        pltpu.VMEM((T,), jnp.int32),
        pltpu.VMEM((T,), jnp.float32),
        pltpu.VMEM((T,), jnp.float32),
        pltpu.VMEM((T,), jnp.float32),
        pltpu.SemaphoreType.DMA,
        pltpu.SemaphoreType.DMA,
        pltpu.SemaphoreType.DMA,
        pltpu.SemaphoreType.DMA,
        pltpu.VMEM_SHARED((NS * ACC_W,), jnp.float32),
    ],
  )(_sc_scatter_body)
  gather = functools.partial(
    pl.kernel,
    mesh=mesh,
    out_type=[jax.ShapeDtypeStruct((B * T,), jnp.float32)],
    scratch_types=[
        pltpu.VMEM((T,), jnp.int32),
        pltpu.VMEM((T,), jnp.int32),
        pltpu.VMEM((T,), jnp.int32),
        pltpu.VMEM((V,), jnp.float32),
        pltpu.VMEM((T,), jnp.float32),
        pltpu.VMEM((T,), jnp.float32),
        pltpu.SemaphoreType.DMA,
        pltpu.SemaphoreType.DMA,
        pltpu.SemaphoreType.DMA,
        pltpu.SemaphoreType.DMA,
        pltpu.SemaphoreType.DMA,
        pltpu.SemaphoreType.DMA,
        pltpu.VMEM_SHARED((NS * V,), jnp.float32),
    ],
  )(_sc_gather_body)
  return scatter, gather


def _vocab_topk_body(x_ref, vo_ref, io_ref):
    x = x_ref[...].reshape(RB, NCH, 128)
    chunk = lax.broadcasted_iota(jnp.int32, (RB, NCH, 128), 1)
    lane3 = lax.broadcasted_iota(jnp.int32, (RB, NCH, 128), 2)
    lane2 = lax.broadcasted_iota(jnp.int32, (RB, 128), 1)
    x = jnp.where(chunk * 128 + lane3 < V, x, -1.0)
    x = jnp.concatenate([x, jnp.full((RB, NCH4 - NCH, 128), -1.0)], axis=1)

    x4v = x.reshape(RB, NG, 4, 128)
    s = [x4v[:, :, j, :] for j in range(4)]
    si = [jnp.full((RB, NG, 128), j, jnp.int32) for j in range(4)]
    for a, bq in ((0, 1), (2, 3), (0, 2), (1, 3), (1, 2)):
        win = (s[a] > s[bq]) | ((s[a] == s[bq]) & (si[a] < si[bq]))
        s[a], s[bq] = (jnp.where(win, s[a], s[bq]),
                       jnp.where(win, s[bq], s[a]))
        si[a], si[bq] = (jnp.where(win, si[a], si[bq]),
                         jnp.where(win, si[bq], si[a]))
    grp = lax.broadcasted_iota(jnp.int32, (RB, NG, 128), 1)
    lane4 = lax.broadcasted_iota(jnp.int32, (RB, NG, 128), 2)

    cand_v = []
    cand_i = []
    for j, kk in ((0, K), (1, 4), (2, 2), (3, 2)):
        v = s[j]
        g = (grp * 4 + si[j]) * 128 + lane4
        for t in range(kk):
            m = jnp.max(v, axis=1, keepdims=True)
            pos = jnp.where(v == m, g, BIGID)
            am = jnp.min(pos, axis=1, keepdims=True)
            cand_v.append(m)
            cand_i.append(am)
            if t + 1 < kk:
                v = jnp.where(g == am, -2.0, v)
    cv = jnp.concatenate(cand_v, axis=1)
    ci = jnp.concatenate(cand_i, axis=1)

    vo = jnp.full((RB, 128), NEG, jnp.float32)
    io = jnp.full((RB, 128), BIGID, jnp.int32)
    for t in range(K):
        m2 = jnp.max(cv, axis=2, keepdims=True)
        m = jnp.max(m2, axis=1, keepdims=True)
        idc = jnp.where(cv == m, ci, BIGID)
        mid2 = jnp.min(idc, axis=2, keepdims=True)
        mid = jnp.min(mid2, axis=1, keepdims=True)
        vo = jnp.where(lane2 == t, m[:, 0, :], vo)
        io = jnp.where(lane2 == t, mid[:, 0, :], io)
        cv = jnp.where((cv == m) & (ci == mid), NEG, cv)
    vo_ref[...] = vo
    io_ref[...] = io


_k1_call = pl.pallas_call(
    _vocab_topk_body,
    grid=(B // RB,),
    in_specs=[pl.BlockSpec((RB, VP), lambda i: (i, 0))],
    out_specs=[
        pl.BlockSpec((RB, 128), lambda i: (i, 0)),
        pl.BlockSpec((RB, 128), lambda i: (i, 0)),
    ],
    out_shape=[
        jax.ShapeDtypeStruct((B, 128), jnp.float32),
        jax.ShapeDtypeStruct((B, 128), jnp.int32),
    ],
)


def _merge_body(t_ref, g_ref, id_ref, p_ref, vv_ref, vi_ref, ov_ref, oi_ref):
    p = p_ref[...][:, :1]
    tv = t_ref[...]
    ids = id_ref[...]
    gv = jnp.where(ids < V, g_ref[...], 0.0)
    sc_val = (1.0 - p) * tv + p * gv

    vt = p * vv_ref[...]
    cv = jnp.concatenate([sc_val.reshape(B, 16, 128), vt[:, None, :]], axis=1)
    ci = jnp.concatenate([ids.reshape(B, 16, 128), vi_ref[...][:, None, :]],
                         axis=1)
    lane2 = lax.broadcasted_iota(jnp.int32, (B, 128), 1)

    ov = jnp.full((B, 128), 0.0, jnp.float32)
    oi = jnp.full((B, 128), 0, jnp.int32)
    for t in range(K):
        m2 = jnp.max(cv, axis=2, keepdims=True)
        m = jnp.max(m2, axis=1, keepdims=True)
        idc = jnp.where(cv == m, ci, BIGID)
        mid2 = jnp.min(idc, axis=2, keepdims=True)
        mid = jnp.min(mid2, axis=1, keepdims=True)
        ov = jnp.where(lane2 == t, jnp.log(m[:, 0, :] + 1e-10), ov)
        oi = jnp.where(lane2 == t, mid[:, 0, :], oi)
        cv = jnp.where(ci == mid, NEG, cv)
    ov_ref[...] = ov
    oi_ref[...] = oi


_k2_call = pl.pallas_call(
    _merge_body,
    grid=(1,),
    in_specs=[
        pl.BlockSpec((B, T), lambda i: (0, 0)),
        pl.BlockSpec((B, T), lambda i: (0, 0)),
        pl.BlockSpec((B, T), lambda i: (0, 0)),
        pl.BlockSpec((B, 128), lambda i: (0, 0)),
        pl.BlockSpec((B, 128), lambda i: (0, 0)),
        pl.BlockSpec((B, 128), lambda i: (0, 0)),
    ],
    out_specs=[
        pl.BlockSpec((B, 128), lambda i: (0, 0)),
        pl.BlockSpec((B, 128), lambda i: (0, 0)),
    ],
    out_shape=[
        jax.ShapeDtypeStruct((B, 128), jnp.float32),
        jax.ShapeDtypeStruct((B, 128), jnp.int32),
    ],
)


def kernel(vocab_dists, attn_dists, p_gens, input_ids):
    ids = input_ids.astype(jnp.int32)
    sc_scatter, sc_gather = _make_sc_calls()
    ids_flat = ids.reshape(-1)
    t3 = sc_scatter(attn_dists.reshape(-1), ids_flat)
    if isinstance(t3, (tuple, list)):
        t3 = t3[0]
    g3 = sc_gather(vocab_dists, ids_flat)
    if isinstance(g3, (tuple, list)):
        g3 = g3[0]
    vv, vi = _k1_call(vocab_dists)
    pbc = jnp.broadcast_to(p_gens, (B, 128))
    lv, li = _k2_call(t3.reshape(B, T), g3.reshape(B, T), ids, pbc, vv, vi)
    return lv[:, :K], li[:, :K]


# speedup vs baseline: 2.4540x; 2.4540x over previous
import functools

import jax
import jax.numpy as jnp
from jax import lax
from jax.experimental import pallas as pl
from jax.experimental.pallas import tpu as pltpu
from jax.experimental.pallas import tpu_sc as plsc

V = 50000
EXT = 51000
B = 128
T = 2048
K = 8
VP = 50048
NCH = VP // 128
NG = 97
RB = 8
NEG = -1e30
BIGID = 2**30

NC, NS = 2, 16
NW = NC * NS
RPW = B // NW
ACC_W = 51200


def _sc_scatter_body(attn_hbm, ids_hbm, t_hbm,
                     ids0, ids1, attn0, attn1, loc_v, t0, t1, z_v,
                     sem_in0, sem_in1, sem_o0, sem_o1, acc):
    ids_v = (ids0, ids1)
    attn_v = (attn0, attn1)
    t_v = (t0, t1)
    sem_in = (sem_in0, sem_in1)
    sem_o = (sem_o0, sem_o1)
    cid = lax.axis_index("c")
    sid = lax.axis_index("s")
    wid = sid * NC + cid
    base = sid * ACC_W

    def zf1(k, c):
        z_v[pl.ds(k * 16, 16)] = jnp.zeros((16,), jnp.float32)
        return c
    lax.fori_loop(0, T // 16, zf1, 0)

    def roff(i):
        return pl.multiple_of((wid * RPW + i) * T, T)

    pend_in = [
        (pltpu.async_copy(ids_hbm.at[pl.ds(roff(0), T)], ids_v[0], sem_in[0]),
         pltpu.async_copy(attn_hbm.at[pl.ds(roff(0), T)], attn_v[0], sem_in[0])),
        None,
    ]
    pend_out = [None, None]

    for i in range(RPW):
        b = i % 2
        for cp in pend_in[b]:
            cp.wait()
        if i + 1 < RPW:
            pend_in[1 - b] = (
                pltpu.async_copy(ids_hbm.at[pl.ds(roff(i + 1), T)],
                                 ids_v[1 - b], sem_in[1 - b]),
                pltpu.async_copy(attn_hbm.at[pl.ds(roff(i + 1), T)],
                                 attn_v[1 - b], sem_in[1 - b]),
            )
        if pend_out[b] is not None:
            pend_out[b].wait()

        def ck(k, cc):
            sl = pl.ds(k * 16, 16)
            loc_v[sl] = ids_v[b][sl] + base
            return cc
        lax.fori_loop(0, T // 16, ck, 0)

        pltpu.sync_copy(z_v, acc.at[loc_v])
        pltpu.sync_copy(attn_v[b], acc.at[loc_v], add=True)
        pltpu.sync_copy(acc.at[loc_v], t_v[b])
        pend_out[b] = pltpu.async_copy(t_v[b], t_hbm.at[pl.ds(roff(i), T)],
                                       sem_o[b])

    for po in pend_out:
        if po is not None:
            po.wait()


def _sc_gather_body(vocab_hbm, ids_hbm, g_hbm,
                    ids0, ids1, gid_v, vtile, g0, g1,
                    sem_in0, sem_in1, sem_vt, sem_g, sem_o0, sem_o1, vrowS):
    ids_v = (ids0, ids1)
    g_v = (g0, g1)
    sem_in = (sem_in0, sem_in1)
    sem_o = (sem_o0, sem_o1)
    cid = lax.axis_index("c")
    sid = lax.axis_index("s")
    wid = sid * NC + cid
    vbase = pl.multiple_of(sid * V, 8)

    def roff(i):
        return pl.multiple_of((wid * RPW + i) * T, T)

    pend_in = [
        pltpu.async_copy(ids_hbm.at[pl.ds(roff(0), T)], ids_v[0], sem_in[0]),
        None,
    ]
    pend_out = [None, None]
    pend_vt = pltpu.async_copy(vocab_hbm.at[wid * RPW], vtile, sem_vt)

    for i in range(RPW):
        b = i % 2
        pend_in[b].wait()
        if i + 1 < RPW:
            pend_in[1 - b] = pltpu.async_copy(
                ids_hbm.at[pl.ds(roff(i + 1), T)], ids_v[1 - b], sem_in[1 - b])

        def ck(k, cc):
            sl = pl.ds(k * 16, 16)
            gid_v[sl] = jnp.minimum(ids_v[b][sl], V - 1) + vbase
            return cc
        lax.fori_loop(0, T // 16, ck, 0)

        pend_vt.wait()
        pltpu.sync_copy(vtile, vrowS.at[pl.ds(vbase, V)])
        if i + 1 < RPW:
            pend_vt = pltpu.async_copy(vocab_hbm.at[wid * RPW + i + 1],
                                       vtile, sem_vt)
        if pend_out[b] is not None:
            pend_out[b].wait()
        pltpu.sync_copy(vrowS.at[gid_v], g_v[b])
        pend_out[b] = pltpu.async_copy(g_v[b], g_hbm.at[pl.ds(roff(i), T)],
                                       sem_o[b])

    for po in pend_out:
        if po is not None:
            po.wait()


@functools.lru_cache(maxsize=1)
def _make_sc_calls():
  mesh = plsc.VectorSubcoreMesh(core_axis_name="c", subcore_axis_name="s")
  scatter = functools.partial(
    pl.kernel,
    mesh=mesh,
    out_type=[jax.ShapeDtypeStruct((B * T,), jnp.float32)],
    scratch_types=[
        pltpu.VMEM((T,), jnp.int32),
        pltpu.VMEM((T,), jnp.int32),
        pltpu.VMEM((T,), jnp.float32),
        pltpu.VMEM((T,), jnp.float32),
        pltpu.VMEM((T,), jnp.int32),
        pltpu.VMEM((T,), jnp.float32),
        pltpu.VMEM((T,), jnp.float32),
        pltpu.VMEM((T,), jnp.float32),
        pltpu.SemaphoreType.DMA,
        pltpu.SemaphoreType.DMA,
        pltpu.SemaphoreType.DMA,
        pltpu.SemaphoreType.DMA,
        pltpu.VMEM_SHARED((NS * ACC_W,), jnp.float32),
    ],
  )(_sc_scatter_body)
  gather = functools.partial(
    pl.kernel,
    mesh=mesh,
    out_type=[jax.ShapeDtypeStruct((B * T,), jnp.float32)],
    scratch_types=[
        pltpu.VMEM((T,), jnp.int32),
        pltpu.VMEM((T,), jnp.int32),
        pltpu.VMEM((T,), jnp.int32),
        pltpu.VMEM((V,), jnp.float32),
        pltpu.VMEM((T,), jnp.float32),
        pltpu.VMEM((T,), jnp.float32),
        pltpu.SemaphoreType.DMA,
        pltpu.SemaphoreType.DMA,
        pltpu.SemaphoreType.DMA,
        pltpu.SemaphoreType.DMA,
        pltpu.SemaphoreType.DMA,
        pltpu.SemaphoreType.DMA,
        pltpu.VMEM_SHARED((NS * V,), jnp.float32),
    ],
  )(_sc_gather_body)
  return scatter, gather


def _vocab_topk_body(x_ref, vo_ref, io_ref):
    x = x_ref[...].reshape(RB, NCH, 128)
    chunk = lax.broadcasted_iota(jnp.int32, (RB, NCH, 128), 1)
    lane3 = lax.broadcasted_iota(jnp.int32, (RB, NCH, 128), 2)
    lane2 = lax.broadcasted_iota(jnp.int32, (RB, 128), 1)
    x = jnp.where(chunk * 128 + lane3 < V, x, -1.0)

    s = [x[:, j * NG:(j + 1) * NG, :] for j in range(4)]
    si = [jnp.full((RB, NG, 128), j, jnp.int32) for j in range(4)]
    for a, bq in ((0, 1), (2, 3), (0, 2), (1, 3), (1, 2)):
        win = (s[a] > s[bq]) | ((s[a] == s[bq]) & (si[a] < si[bq]))
        s[a], s[bq] = (jnp.where(win, s[a], s[bq]),
                       jnp.where(win, s[bq], s[a]))
        si[a], si[bq] = (jnp.where(win, si[a], si[bq]),
                         jnp.where(win, si[bq], si[a]))
    grp = lax.broadcasted_iota(jnp.int32, (RB, NG, 128), 1)
    lane4 = lax.broadcasted_iota(jnp.int32, (RB, NG, 128), 2)

    cand_v = []
    cand_i = []
    for j, kk in ((0, K), (1, 4), (2, 2), (3, 2)):
        v = s[j]
        g = (si[j] * NG + grp) * 128 + lane4
        for t in range(kk):
            m = jnp.max(v, axis=1, keepdims=True)
            pos = jnp.where(v == m, g, BIGID)
            am = jnp.min(pos, axis=1, keepdims=True)
            cand_v.append(m)
            cand_i.append(am)
            if t + 1 < kk:
                v = jnp.where(g == am, -2.0, v)
    for t in range(4 * NG, NCH):
        cand_v.append(x[:, t:t + 1, :])
        cand_i.append(chunk[:, t:t + 1, :] * 128 + lane3[:, t:t + 1, :])
    cv = jnp.concatenate(cand_v, axis=1)
    ci = jnp.concatenate(cand_i, axis=1)

    vo = jnp.full((RB, 128), NEG, jnp.float32)
    io = jnp.full((RB, 128), BIGID, jnp.int32)
    for t in range(K):
        m2 = jnp.max(cv, axis=2, keepdims=True)
        m = jnp.max(m2, axis=1, keepdims=True)
        idc = jnp.where(cv == m, ci, BIGID)
        mid2 = jnp.min(idc, axis=2, keepdims=True)
        mid = jnp.min(mid2, axis=1, keepdims=True)
        vo = jnp.where(lane2 == t, m[:, 0, :], vo)
        io = jnp.where(lane2 == t, mid[:, 0, :], io)
        cv = jnp.where((cv == m) & (ci == mid), NEG, cv)
    vo_ref[...] = vo
    io_ref[...] = io


_k1_call = pl.pallas_call(
    _vocab_topk_body,
    grid=(B // RB,),
    in_specs=[pl.BlockSpec((RB, VP), lambda i: (i, 0))],
    out_specs=[
        pl.BlockSpec((RB, 128), lambda i: (i, 0)),
        pl.BlockSpec((RB, 128), lambda i: (i, 0)),
    ],
    out_shape=[
        jax.ShapeDtypeStruct((B, 128), jnp.float32),
        jax.ShapeDtypeStruct((B, 128), jnp.int32),
    ],
)


def _merge_body(t_ref, g_ref, id_ref, p_ref, vv_ref, vi_ref, ov_ref, oi_ref):
    p = p_ref[...][:, :1]
    tv = t_ref[...]
    ids = id_ref[...]
    gv = jnp.where(ids < V, g_ref[...], 0.0)
    sc_val = (1.0 - p) * tv + p * gv

    vt = p * vv_ref[...]
    cv = jnp.concatenate([sc_val.reshape(B, 16, 128), vt[:, None, :]], axis=1)
    ci = jnp.concatenate([ids.reshape(B, 16, 128), vi_ref[...][:, None, :]],
                         axis=1)
    lane2 = lax.broadcasted_iota(jnp.int32, (B, 128), 1)

    ov = jnp.full((B, 128), 0.0, jnp.float32)
    oi = jnp.full((B, 128), 0, jnp.int32)
    for t in range(K):
        m2 = jnp.max(cv, axis=2, keepdims=True)
        m = jnp.max(m2, axis=1, keepdims=True)
        idc = jnp.where(cv == m, ci, BIGID)
        mid2 = jnp.min(idc, axis=2, keepdims=True)
        mid = jnp.min(mid2, axis=1, keepdims=True)
        ov = jnp.where(lane2 == t, jnp.log(m[:, 0, :] + 1e-10), ov)
        oi = jnp.where(lane2 == t, mid[:, 0, :], oi)
        cv = jnp.where(ci == mid, NEG, cv)
    ov_ref[...] = ov
    oi_ref[...] = oi


_k2_call = pl.pallas_call(
    _merge_body,
    grid=(1,),
    in_specs=[
        pl.BlockSpec((B, T), lambda i: (0, 0)),
        pl.BlockSpec((B, T), lambda i: (0, 0)),
        pl.BlockSpec((B, T), lambda i: (0, 0)),
        pl.BlockSpec((B, 128), lambda i: (0, 0)),
        pl.BlockSpec((B, 128), lambda i: (0, 0)),
        pl.BlockSpec((B, 128), lambda i: (0, 0)),
    ],
    out_specs=[
        pl.BlockSpec((B, 128), lambda i: (0, 0)),
        pl.BlockSpec((B, 128), lambda i: (0, 0)),
    ],
    out_shape=[
        jax.ShapeDtypeStruct((B, 128), jnp.float32),
        jax.ShapeDtypeStruct((B, 128), jnp.int32),
    ],
)


def kernel(vocab_dists, attn_dists, p_gens, input_ids):
    ids = input_ids.astype(jnp.int32)
    sc_scatter, sc_gather = _make_sc_calls()
    ids_flat = ids.reshape(-1)
    t3 = sc_scatter(attn_dists.reshape(-1), ids_flat)
    if isinstance(t3, (tuple, list)):
        t3 = t3[0]
    g3 = sc_gather(vocab_dists, ids_flat)
    if isinstance(g3, (tuple, list)):
        g3 = g3[0]
    vv, vi = _k1_call(vocab_dists)
    pbc = jnp.broadcast_to(p_gens, (B, 128))
    lv, li = _k2_call(t3.reshape(B, T), g3.reshape(B, T), ids, pbc, vv, vi)
    return lv[:, :K], li[:, :K]


# speedup vs baseline: 2.4808x; 1.0109x over previous
import functools

import jax
import jax.numpy as jnp
from jax import lax
from jax.experimental import pallas as pl
from jax.experimental.pallas import tpu as pltpu
from jax.experimental.pallas import tpu_sc as plsc

V = 50000
EXT = 51000
B = 128
T = 2048
K = 8
VP = 50048
NCH = VP // 128
NG = 97
RB = 8
NEG = -1e30
BIGID = 2**30

NC, NS = 2, 16
NW = NC * NS
RPW = B // NW
ACC_W = 51200


def _sc_scatter_body(attn_hbm, ids_hbm, t_hbm,
                     ids0, ids1, attn0, attn1, loc_v, t0, t1, z_v,
                     sem_in0, sem_in1, sem_o0, sem_o1, acc):
    ids_v = (ids0, ids1)
    attn_v = (attn0, attn1)
    t_v = (t0, t1)
    sem_in = (sem_in0, sem_in1)
    sem_o = (sem_o0, sem_o1)
    cid = lax.axis_index("c")
    sid = lax.axis_index("s")
    wid = sid * NC + cid
    base = sid * ACC_W

    def zf1(k, c):
        z_v[pl.ds(k * 16, 16)] = jnp.zeros((16,), jnp.float32)
        return c
    lax.fori_loop(0, T // 16, zf1, 0)

    def roff(i):
        return pl.multiple_of((wid * RPW + i) * T, T)

    pend_in = [
        (pltpu.async_copy(ids_hbm.at[pl.ds(roff(0), T)], ids_v[0], sem_in[0]),
         pltpu.async_copy(attn_hbm.at[pl.ds(roff(0), T)], attn_v[0], sem_in[0])),
        None,
    ]
    pend_out = [None, None]

    for i in range(RPW):
        b = i % 2
        for cp in pend_in[b]:
            cp.wait()
        if i + 1 < RPW:
            pend_in[1 - b] = (
                pltpu.async_copy(ids_hbm.at[pl.ds(roff(i + 1), T)],
                                 ids_v[1 - b], sem_in[1 - b]),
                pltpu.async_copy(attn_hbm.at[pl.ds(roff(i + 1), T)],
                                 attn_v[1 - b], sem_in[1 - b]),
            )
        if pend_out[b] is not None:
            pend_out[b].wait()

        def ck(k, cc):
            sl = pl.ds(k * 16, 16)
            loc_v[sl] = ids_v[b][sl] + base
            return cc
        lax.fori_loop(0, T // 16, ck, 0)

        pltpu.sync_copy(z_v, acc.at[loc_v])
        pltpu.sync_copy(attn_v[b], acc.at[loc_v], add=True)
        pltpu.sync_copy(acc.at[loc_v], t_v[b])
        pend_out[b] = pltpu.async_copy(t_v[b], t_hbm.at[pl.ds(roff(i), T)],
                                       sem_o[b])

    for po in pend_out:
        if po is not None:
            po.wait()


def _sc_gather_body(vocab_hbm, ids_hbm, g_hbm,
                    ids0, ids1, gid_v, vtile, g0, g1,
                    sem_in0, sem_in1, sem_vt, sem_g, sem_o0, sem_o1, vrowS):
    ids_v = (ids0, ids1)
    g_v = (g0, g1)
    sem_in = (sem_in0, sem_in1)
    sem_o = (sem_o0, sem_o1)
    cid = lax.axis_index("c")
    sid = lax.axis_index("s")
    wid = sid * NC + cid
    vbase = pl.multiple_of(sid * V, 8)

    def roff(i):
        return pl.multiple_of((wid * RPW + i) * T, T)

    pend_in = [
        pltpu.async_copy(ids_hbm.at[pl.ds(roff(0), T)], ids_v[0], sem_in[0]),
        None,
    ]
    pend_out = [None, None]
    pend_vt = pltpu.async_copy(vocab_hbm.at[wid * RPW], vtile, sem_vt)

    for i in range(RPW):
        b = i % 2
        pend_in[b].wait()
        if i + 1 < RPW:
            pend_in[1 - b] = pltpu.async_copy(
                ids_hbm.at[pl.ds(roff(i + 1), T)], ids_v[1 - b], sem_in[1 - b])

        def ck(k, cc):
            sl = pl.ds(k * 16, 16)
            gid_v[sl] = jnp.minimum(ids_v[b][sl], V - 1) + vbase
            return cc
        lax.fori_loop(0, T // 16, ck, 0)

        pend_vt.wait()
        pltpu.sync_copy(vtile, vrowS.at[pl.ds(vbase, V)])
        if i + 1 < RPW:
            pend_vt = pltpu.async_copy(vocab_hbm.at[wid * RPW + i + 1],
                                       vtile, sem_vt)
        if pend_out[b] is not None:
            pend_out[b].wait()
        pltpu.sync_copy(vrowS.at[gid_v], g_v[b])
        pend_out[b] = pltpu.async_copy(g_v[b], g_hbm.at[pl.ds(roff(i), T)],
                                       sem_o[b])

    for po in pend_out:
        if po is not None:
            po.wait()


@functools.lru_cache(maxsize=1)
def _make_sc_calls():
  mesh = plsc.VectorSubcoreMesh(core_axis_name="c", subcore_axis_name="s")
  scatter = functools.partial(
    pl.kernel,
    mesh=mesh,
    out_type=[jax.ShapeDtypeStruct((B * T,), jnp.float32)],
    scratch_types=[
        pltpu.VMEM((T,), jnp.int32),
        pltpu.VMEM((T,), jnp.int32),
        pltpu.VMEM((T,), jnp.float32),
        pltpu.VMEM((T,), jnp.float32),
        pltpu.VMEM((T,), jnp.int32),
        pltpu.VMEM((T,), jnp.float32),
        pltpu.VMEM((T,), jnp.float32),
        pltpu.VMEM((T,), jnp.float32),
        pltpu.SemaphoreType.DMA,
        pltpu.SemaphoreType.DMA,
        pltpu.SemaphoreType.DMA,
        pltpu.SemaphoreType.DMA,
        pltpu.VMEM_SHARED((NS * ACC_W,), jnp.float32),
    ],
  )(_sc_scatter_body)
  gather = functools.partial(
    pl.kernel,
    mesh=mesh,
    out_type=[jax.ShapeDtypeStruct((B * T,), jnp.float32)],
    scratch_types=[
        pltpu.VMEM((T,), jnp.int32),
        pltpu.VMEM((T,), jnp.int32),
        pltpu.VMEM((T,), jnp.int32),
        pltpu.VMEM((V,), jnp.float32),
        pltpu.VMEM((T,), jnp.float32),
        pltpu.VMEM((T,), jnp.float32),
        pltpu.SemaphoreType.DMA,
        pltpu.SemaphoreType.DMA,
        pltpu.SemaphoreType.DMA,
        pltpu.SemaphoreType.DMA,
        pltpu.SemaphoreType.DMA,
        pltpu.SemaphoreType.DMA,
        pltpu.VMEM_SHARED((NS * V,), jnp.float32),
    ],
  )(_sc_gather_body)
  return scatter, gather


def _vocab_topk_body(x_ref, vo_ref, io_ref):
    x = x_ref[...].reshape(RB, NCH, 128)
    chunk = lax.broadcasted_iota(jnp.int32, (RB, NCH, 128), 1)
    lane3 = lax.broadcasted_iota(jnp.int32, (RB, NCH, 128), 2)
    lane2 = lax.broadcasted_iota(jnp.int32, (RB, 128), 1)
    x = jnp.where(chunk * 128 + lane3 < V, x, -1.0)

    cand_v = []
    cand_i = []
    for _ in range(K):
        m = jnp.max(x, axis=1, keepdims=True)
        pos = jnp.where(x == m, chunk, NCH)
        am = jnp.min(pos, axis=1, keepdims=True)
        cand_v.append(m)
        cand_i.append(am * 128 + lane3[:, :1, :])
        x = jnp.where(chunk == am, -2.0, x)
    cv = jnp.concatenate(cand_v, axis=1)
    ci = jnp.concatenate(cand_i, axis=1)

    vo = jnp.full((RB, 128), NEG, jnp.float32)
    io = jnp.full((RB, 128), BIGID, jnp.int32)
    for t in range(K):
        m2 = jnp.max(cv, axis=2, keepdims=True)
        m = jnp.max(m2, axis=1, keepdims=True)
        idc = jnp.where(cv == m, ci, BIGID)
        mid2 = jnp.min(idc, axis=2, keepdims=True)
        mid = jnp.min(mid2, axis=1, keepdims=True)
        vo = jnp.where(lane2 == t, m[:, 0, :], vo)
        io = jnp.where(lane2 == t, mid[:, 0, :], io)
        cv = jnp.where((cv == m) & (ci == mid), NEG, cv)
    vo_ref[...] = vo
    io_ref[...] = io


_k1_call = pl.pallas_call(
    _vocab_topk_body,
    grid=(B // RB,),
    in_specs=[pl.BlockSpec((RB, VP), lambda i: (i, 0))],
    out_specs=[
        pl.BlockSpec((RB, 128), lambda i: (i, 0)),
        pl.BlockSpec((RB, 128), lambda i: (i, 0)),
    ],
    out_shape=[
        jax.ShapeDtypeStruct((B, 128), jnp.float32),
        jax.ShapeDtypeStruct((B, 128), jnp.int32),
    ],
)


def _merge_body(t_ref, g_ref, id_ref, p_ref, vv_ref, vi_ref, ov_ref, oi_ref):
    p = p_ref[...][:, :1]
    tv = t_ref[...]
    ids = id_ref[...]
    gv = jnp.where(ids < V, g_ref[...], 0.0)
    sc_val = (1.0 - p) * tv + p * gv

    vt = p * vv_ref[...]
    cv = jnp.concatenate([sc_val.reshape(B, 16, 128), vt[:, None, :]], axis=1)
    ci = jnp.concatenate([ids.reshape(B, 16, 128), vi_ref[...][:, None, :]],
                         axis=1)
    lane2 = lax.broadcasted_iota(jnp.int32, (B, 128), 1)

    ov = jnp.full((B, 128), 0.0, jnp.float32)
    oi = jnp.full((B, 128), 0, jnp.int32)
    for t in range(K):
        m2 = jnp.max(cv, axis=2, keepdims=True)
        m = jnp.max(m2, axis=1, keepdims=True)
        idc = jnp.where(cv == m, ci, BIGID)
        mid2 = jnp.min(idc, axis=2, keepdims=True)
        mid = jnp.min(mid2, axis=1, keepdims=True)
        ov = jnp.where(lane2 == t, jnp.log(m[:, 0, :] + 1e-10), ov)
        oi = jnp.where(lane2 == t, mid[:, 0, :], oi)
        cv = jnp.where(ci == mid, NEG, cv)
    ov_ref[...] = ov
    oi_ref[...] = oi


_k2_call = pl.pallas_call(
    _merge_body,
    grid=(1,),
    in_specs=[
        pl.BlockSpec((B, T), lambda i: (0, 0)),
        pl.BlockSpec((B, T), lambda i: (0, 0)),
        pl.BlockSpec((B, T), lambda i: (0, 0)),
        pl.BlockSpec((B, 128), lambda i: (0, 0)),
        pl.BlockSpec((B, 128), lambda i: (0, 0)),
        pl.BlockSpec((B, 128), lambda i: (0, 0)),
    ],
    out_specs=[
        pl.BlockSpec((B, 128), lambda i: (0, 0)),
        pl.BlockSpec((B, 128), lambda i: (0, 0)),
    ],
    out_shape=[
        jax.ShapeDtypeStruct((B, 128), jnp.float32),
        jax.ShapeDtypeStruct((B, 128), jnp.int32),
    ],
)


def kernel(vocab_dists, attn_dists, p_gens, input_ids):
    ids = input_ids.astype(jnp.int32)
    sc_scatter, sc_gather = _make_sc_calls()
    ids_flat = ids.reshape(-1)
    t3 = sc_scatter(attn_dists.reshape(-1), ids_flat)
    if isinstance(t3, (tuple, list)):
        t3 = t3[0]
    g3 = sc_gather(vocab_dists, ids_flat)
    if isinstance(g3, (tuple, list)):
        g3 = g3[0]
    vv, vi = _k1_call(vocab_dists)
    pbc = jnp.broadcast_to(p_gens, (B, 128))
    lv, li = _k2_call(t3.reshape(B, T), g3.reshape(B, T), ids, pbc, vv, vi)
    return lv[:, :K], li[:, :K]


# speedup vs baseline: 2.5140x; 1.0134x over previous
import functools

import jax
import jax.numpy as jnp
from jax import lax
from jax.experimental import pallas as pl
from jax.experimental.pallas import tpu as pltpu
from jax.experimental.pallas import tpu_sc as plsc

V = 50000
EXT = 51000
B = 128
T = 2048
K = 8
VP = 50048
NCH = VP // 128
NG = 97
RB = 8
NEG = -1e30
BIGID = 2**30

NC, NS = 2, 16
NW = NC * NS
RPW = B // NW
ACC_W = 51200


def _sc_body(vocab_hbm, attn_hbm, ids_hbm, t_hbm, g_hbm,
             ids0, ids1, attn0, attn1, loc_v, gid_v, vtile, t0, t1, g0, g1,
             z_v, sem_in0, sem_in1, sem_vt, sem_o0, sem_o1, acc):
    ids_v = (ids0, ids1)
    attn_v = (attn0, attn1)
    t_v = (t0, t1)
    g_v = (g0, g1)
    sem_in = (sem_in0, sem_in1)
    sem_o = (sem_o0, sem_o1)
    cid = lax.axis_index("c")
    sid = lax.axis_index("s")
    wid = sid * NC + cid
    base = sid * ACC_W

    def zf1(k, c):
        z_v[pl.ds(k * 16, 16)] = jnp.zeros((16,), jnp.float32)
        return c
    lax.fori_loop(0, T // 16, zf1, 0)

    def roff(i):
        return pl.multiple_of((wid * RPW + i) * T, T)

    pend_in = [
        (pltpu.async_copy(ids_hbm.at[pl.ds(roff(0), T)], ids_v[0], sem_in[0]),
         pltpu.async_copy(attn_hbm.at[pl.ds(roff(0), T)], attn_v[0], sem_in[0])),
        None,
    ]
    pend_out = [None, None]
    pend_vt = pltpu.async_copy(vocab_hbm.at[wid * RPW], vtile, sem_vt)

    for i in range(RPW):
        b = i % 2
        for cp in pend_in[b]:
            cp.wait()
        if i + 1 < RPW:
            pend_in[1 - b] = (
                pltpu.async_copy(ids_hbm.at[pl.ds(roff(i + 1), T)],
                                 ids_v[1 - b], sem_in[1 - b]),
                pltpu.async_copy(attn_hbm.at[pl.ds(roff(i + 1), T)],
                                 attn_v[1 - b], sem_in[1 - b]),
            )
        if pend_out[b] is not None:
            for cp in pend_out[b]:
                cp.wait()

        def ck(k, cc):
            sl = pl.ds(k * 16, 16)
            idv = ids_v[b][sl]
            loc_v[sl] = idv + base
            gid_v[sl] = jnp.minimum(idv, V - 1) + base
            return cc
        lax.fori_loop(0, T // 16, ck, 0)

        pltpu.sync_copy(z_v, acc.at[loc_v])
        pltpu.sync_copy(attn_v[b], acc.at[loc_v], add=True)
        pltpu.sync_copy(acc.at[loc_v], t_v[b])
        wt = pltpu.async_copy(t_v[b], t_hbm.at[pl.ds(roff(i), T)], sem_o[b])

        pend_vt.wait()
        pltpu.sync_copy(vtile, acc.at[pl.ds(base, V)])
        if i + 1 < RPW:
            pend_vt = pltpu.async_copy(vocab_hbm.at[wid * RPW + i + 1],
                                       vtile, sem_vt)
        pltpu.sync_copy(acc.at[gid_v], g_v[b])
        wg = pltpu.async_copy(g_v[b], g_hbm.at[pl.ds(roff(i), T)], sem_o[b])
        pend_out[b] = (wt, wg)

    for po in pend_out:
        if po is not None:
            for cp in po:
                cp.wait()


@functools.lru_cache(maxsize=1)
def _make_sc_call():
  return functools.partial(
    pl.kernel,
    mesh=plsc.VectorSubcoreMesh(core_axis_name="c", subcore_axis_name="s"),
    out_type=[
        jax.ShapeDtypeStruct((B * T,), jnp.float32),
        jax.ShapeDtypeStruct((B * T,), jnp.float32),
    ],
    scratch_types=[
        pltpu.VMEM((T,), jnp.int32),
        pltpu.VMEM((T,), jnp.int32),
        pltpu.VMEM((T,), jnp.float32),
        pltpu.VMEM((T,), jnp.float32),
        pltpu.VMEM((T,), jnp.int32),
        pltpu.VMEM((T,), jnp.int32),
        pltpu.VMEM((V,), jnp.float32),
        pltpu.VMEM((T,), jnp.float32),
        pltpu.VMEM((T,), jnp.float32),
        pltpu.VMEM((T,), jnp.float32),
        pltpu.VMEM((T,), jnp.float32),
        pltpu.VMEM((T,), jnp.float32),
        pltpu.SemaphoreType.DMA,
        pltpu.SemaphoreType.DMA,
        pltpu.SemaphoreType.DMA,
        pltpu.SemaphoreType.DMA,
        pltpu.SemaphoreType.DMA,
        pltpu.VMEM_SHARED((NS * ACC_W,), jnp.float32),
    ],
  )(_sc_body)


def _vocab_topk_body(x_ref, vo_ref, io_ref):
    x = x_ref[...].reshape(RB, NCH, 128)
    chunk = lax.broadcasted_iota(jnp.int32, (RB, NCH, 128), 1)
    lane3 = lax.broadcasted_iota(jnp.int32, (RB, NCH, 128), 2)
    lane2 = lax.broadcasted_iota(jnp.int32, (RB, 128), 1)
    x = jnp.where(chunk * 128 + lane3 < V, x, -1.0)

    cand_v = []
    cand_i = []
    for _ in range(K):
        m = jnp.max(x, axis=1, keepdims=True)
        pos = jnp.where(x == m, chunk, NCH)
        am = jnp.min(pos, axis=1, keepdims=True)
        cand_v.append(m)
        cand_i.append(am * 128 + lane3[:, :1, :])
        x = jnp.where(chunk == am, -2.0, x)
    cv = jnp.concatenate(cand_v, axis=1)
    ci = jnp.concatenate(cand_i, axis=1)

    vo = jnp.full((RB, 128), NEG, jnp.float32)
    io = jnp.full((RB, 128), BIGID, jnp.int32)
    for t in range(K):
        m2 = jnp.max(cv, axis=2, keepdims=True)
        m = jnp.max(m2, axis=1, keepdims=True)
        idc = jnp.where(cv == m, ci, BIGID)
        mid2 = jnp.min(idc, axis=2, keepdims=True)
        mid = jnp.min(mid2, axis=1, keepdims=True)
        vo = jnp.where(lane2 == t, m[:, 0, :], vo)
        io = jnp.where(lane2 == t, mid[:, 0, :], io)
        cv = jnp.where((cv == m) & (ci == mid), NEG, cv)
    vo_ref[...] = vo
    io_ref[...] = io


_k1_call = pl.pallas_call(
    _vocab_topk_body,
    grid=(B // RB,),
    in_specs=[pl.BlockSpec((RB, VP), lambda i: (i, 0))],
    out_specs=[
        pl.BlockSpec((RB, 128), lambda i: (i, 0)),
        pl.BlockSpec((RB, 128), lambda i: (i, 0)),
    ],
    out_shape=[
        jax.ShapeDtypeStruct((B, 128), jnp.float32),
        jax.ShapeDtypeStruct((B, 128), jnp.int32),
    ],
)


def _merge_body(t_ref, g_ref, id_ref, p_ref, vv_ref, vi_ref, ov_ref, oi_ref):
    p = p_ref[...][:, :1]
    tv = t_ref[...]
    ids = id_ref[...]
    gv = jnp.where(ids < V, g_ref[...], 0.0)
    sc_val = (1.0 - p) * tv + p * gv

    vt = p * vv_ref[...]
    cv = jnp.concatenate([sc_val.reshape(B, 16, 128), vt[:, None, :]], axis=1)
    ci = jnp.concatenate([ids.reshape(B, 16, 128), vi_ref[...][:, None, :]],
                         axis=1)
    lane2 = lax.broadcasted_iota(jnp.int32, (B, 128), 1)

    ov = jnp.full((B, 128), 0.0, jnp.float32)
    oi = jnp.full((B, 128), 0, jnp.int32)
    for t in range(K):
        m2 = jnp.max(cv, axis=2, keepdims=True)
        m = jnp.max(m2, axis=1, keepdims=True)
        idc = jnp.where(cv == m, ci, BIGID)
        mid2 = jnp.min(idc, axis=2, keepdims=True)
        mid = jnp.min(mid2, axis=1, keepdims=True)
        ov = jnp.where(lane2 == t, jnp.log(m[:, 0, :] + 1e-10), ov)
        oi = jnp.where(lane2 == t, mid[:, 0, :], oi)
        cv = jnp.where(ci == mid, NEG, cv)
    ov_ref[...] = ov
    oi_ref[...] = oi


_k2_call = pl.pallas_call(
    _merge_body,
    grid=(1,),
    in_specs=[
        pl.BlockSpec((B, T), lambda i: (0, 0)),
        pl.BlockSpec((B, T), lambda i: (0, 0)),
        pl.BlockSpec((B, T), lambda i: (0, 0)),
        pl.BlockSpec((B, 128), lambda i: (0, 0)),
        pl.BlockSpec((B, 128), lambda i: (0, 0)),
        pl.BlockSpec((B, 128), lambda i: (0, 0)),
    ],
    out_specs=[
        pl.BlockSpec((B, 128), lambda i: (0, 0)),
        pl.BlockSpec((B, 128), lambda i: (0, 0)),
    ],
    out_shape=[
        jax.ShapeDtypeStruct((B, 128), jnp.float32),
        jax.ShapeDtypeStruct((B, 128), jnp.int32),
    ],
)


def kernel(vocab_dists, attn_dists, p_gens, input_ids):
    ids = input_ids.astype(jnp.int32)
    t3, g3 = _make_sc_call()(vocab_dists, attn_dists.reshape(-1),
                             ids.reshape(-1))
    vv, vi = _k1_call(vocab_dists)
    pbc = jnp.broadcast_to(p_gens, (B, 128))
    lv, li = _k2_call(t3.reshape(B, T), g3.reshape(B, T), ids, pbc, vv, vi)
    return lv[:, :K], li[:, :K]
